# baseline (speedup 1.0000x reference)
import functools

import jax
import jax.numpy as jnp
import numpy as np
from jax.experimental import pallas as pl
from jax.experimental.pallas import tpu as pltpu

N_PTS = 32
DIM_LOI = 128
DIM_FC = 1024
TOPK_JUNC = 300
TOPK_LINE = 5000
H = 256
W = 256
LINES_PAD = 5120


def _make_lambda2():
    order = 1
    p = np.array([1.0, 1.0])
    k = np.arange(0, order + 1)
    t0 = np.linspace(0, 1, order + 1)[:, None]
    coeff0 = p * t0 ** k * (1 - t0) ** (order - k)
    t = np.linspace(0, 1, N_PTS)[:, None]
    coeff = p * t ** k * (1 - t) ** (order - k)
    lam2 = np.matmul(coeff, np.linalg.inv(coeff0))
    return jnp.asarray(lam2, dtype=jnp.float32)

_LAM2 = _make_lambda2()


def _nms_body(x_ref, o_ref):
    h = x_ref[...]
    zr = jnp.zeros((1, W), jnp.float32)
    up = jnp.concatenate([h[1:], zr], axis=0)
    dn = jnp.concatenate([zr, h[:-1]], axis=0)
    m = jnp.maximum(jnp.maximum(h, up), dn)
    zc = jnp.zeros((H, 1), jnp.float32)
    lf = jnp.concatenate([m[:, 1:], zc], axis=1)
    rt = jnp.concatenate([zc, m[:, :-1]], axis=1)
    m = jnp.maximum(jnp.maximum(m, lf), rt)
    o_ref[...] = jnp.where(h == m, h, 0.0)


def _nms_pallas(heat):
    return pl.pallas_call(
        _nms_body,
        out_shape=jax.ShapeDtypeStruct((H, W), jnp.float32),
    )(heat)


def _mlp_body(x_ref, keep_ref, w1_ref, b1_ref, g1_ref, t1_ref,
              w2_ref, b2_ref, g2_ref, t2_ref, w3_ref, b3_ref, o_ref):
    x = x_ref[...]
    h1 = jnp.dot(x, w1_ref[...], preferred_element_type=jnp.float32)
    h1 = jnp.maximum((h1 + b1_ref[...]) * g1_ref[...] + t1_ref[...], 0.0)
    h2 = jnp.dot(h1, w2_ref[...], preferred_element_type=jnp.float32)
    h2 = jnp.maximum((h2 + b2_ref[...]) * g2_ref[...] + t2_ref[...], 0.0)
    s = jnp.dot(h2, w3_ref[...], preferred_element_type=jnp.float32)
    s = jax.nn.sigmoid(s + b3_ref[...])
    o_ref[...] = s * keep_ref[...]


def _mlp_pallas(f, keep, W1, b1, g1, bt1, W2, b2, g2, bt2, W3, b3):
    rows = 512
    grid = LINES_PAD // rows
    w3p = jnp.zeros((DIM_FC, 128), jnp.float32).at[:, 0].set(W3[:, 0])
    b3p = jnp.zeros((1, 128), jnp.float32).at[0, 0].set(b3[0])
    full = lambda i: (0, 0)
    out = pl.pallas_call(
        _mlp_body,
        grid=(grid,),
        in_specs=[
            pl.BlockSpec((rows, DIM_FC), lambda i: (i, 0)),
            pl.BlockSpec((rows, 1), lambda i: (i, 0)),
            pl.BlockSpec((DIM_FC, DIM_FC), full),
            pl.BlockSpec((1, DIM_FC), full),
            pl.BlockSpec((1, DIM_FC), full),
            pl.BlockSpec((1, DIM_FC), full),
            pl.BlockSpec((DIM_FC, DIM_FC), full),
            pl.BlockSpec((1, DIM_FC), full),
            pl.BlockSpec((1, DIM_FC), full),
            pl.BlockSpec((1, DIM_FC), full),
            pl.BlockSpec((DIM_FC, 128), full),
            pl.BlockSpec((1, 128), full),
        ],
        out_specs=pl.BlockSpec((rows, 128), lambda i: (i, 0)),
        out_shape=jax.ShapeDtypeStruct((LINES_PAD, 128), jnp.float32),
    )(f, keep, W1, b1[None], g1[None], bt1[None],
      W2, b2[None], g2[None], bt2[None], w3p, b3p)
    return out


def kernel(outputs, loi_features, W1, b1, g1, bt1, W2, b2, g2, bt2, W3, b3):
    jloc = jax.nn.softmax(outputs[:, 1:3], axis=1)[:, 1:]
    joff = outputs[:, 3:5]
    cloc = jax.nn.softmax(outputs[:, 5:7], axis=1)[:, 1:]
    coff = outputs[:, 7:9]
    eoff = jnp.tanh(outputs[:, 9:]) * 128.0

    heat = _nms_pallas(jloc[0, 0])

    jscore = heat.reshape(-1)
    joff_t = joff[0].reshape(2, -1).T
    _, jidx = jax.lax.top_k(jscore, TOPK_JUNC)
    jy = jidx // H
    jx = jidx % W
    junc = jnp.stack((jx, jy), axis=1).astype(jnp.float32) + joff_t[jidx] + 0.5

    lscore = cloc[0].reshape(-1)
    coff_t = coff[0].reshape(2, -1).T
    eoff_t = jnp.transpose(eoff[0].reshape(2, 2, -1), (2, 1, 0))
    _, lidx = jax.lax.top_k(lscore, TOPK_LINE)
    ly = lidx // H
    lx = lidx % W
    center = jnp.stack((lx, ly), axis=1).astype(jnp.float32) + coff_t[lidx] + 0.5
    line = center[:, None] + eoff_t[lidx]

    d1 = jnp.sum((line[:, None, 0] - junc[None]) ** 2, axis=-1)
    d2 = jnp.sum((line[:, None, -1] - junc[None]) ** 2, axis=-1)
    idx1 = jnp.argmin(d1, axis=1)
    idx2 = jnp.argmin(d2, axis=1)
    idx_min = jnp.minimum(idx1, idx2)
    idx_max = jnp.maximum(idx1, idx2)
    iskeep = (idx_min != idx_max).astype(jnp.float32)
    loi = jnp.stack((junc[idx_min], junc[idx_max]), axis=1)
    swap = loi[:, 0, 1] > loi[:, 1, 1]
    loi = jnp.where(swap[:, None, None], loi[:, ::-1], loi)

    feature = loi_features[0]
    pts = jnp.sum(_LAM2[None, :, :, None] * loi[:, None], axis=2) - 0.5
    pts = pts.reshape(-1, 2)
    px = pts[:, 0]
    py = pts[:, 1]
    px0 = jnp.clip(jnp.floor(px), 0, W - 1)
    py0 = jnp.clip(jnp.floor(py), 0, H - 1)
    px1 = jnp.clip(px0 + 1, 0, W - 1)
    py1 = jnp.clip(py0 + 1, 0, H - 1)
    px0l = px0.astype(jnp.int32)
    py0l = py0.astype(jnp.int32)
    px1l = px1.astype(jnp.int32)
    py1l = py1.astype(jnp.int32)
    f = (feature[:, py0l, px0l] * (py1 - py) * (px1 - px)
         + feature[:, py1l, px0l] * (py - py0) * (px1 - px)
         + feature[:, py0l, px1l] * (py1 - py) * (px - px0)
         + feature[:, py1l, px1l] * (py - py0) * (px - px0))
    f = jnp.transpose(f.reshape(DIM_LOI, -1, N_PTS), (1, 0, 2))
    f = f.reshape(f.shape[0], DIM_LOI, N_PTS // 4, 4).max(axis=-1)
    f = f.reshape(f.shape[0], -1)

    fp = jnp.zeros((LINES_PAD, DIM_FC), jnp.float32).at[:TOPK_LINE].set(f)
    keep = jnp.zeros((LINES_PAD, 1), jnp.float32).at[:TOPK_LINE, 0].set(iskeep)
    out = _mlp_pallas(fp, keep, W1, b1, g1, bt1, W2, b2, g2, bt2, W3, b3)
    return out[:TOPK_LINE, 0]
